# Initial kernel scaffold; baseline (speedup 1.0000x reference)
import jax
import jax.numpy as jnp
from jax.experimental import pallas as pl


def kernel(positions, features, es_W1, es_b1, es_W2, es_b2, ev_W1, ev_b1, ev_W2, ev_b2, le_W1, le_b1, le_W2, le_b2, ln_W1, ln_b1, ln_W2, ln_b2, lv_W1, lv_b1, lv_W2, lv_b2):
    raise NotImplementedError("write your pallas kernel here")


# speedup vs baseline: 88.8751x; 88.8751x over previous
import functools

import jax
import jax.numpy as jnp
from jax.experimental import pallas as pl
from jax.experimental.pallas import tpu as pltpu

K = 16
K2 = 2


def _silu(x):
    return x * jax.nn.sigmoid(x)


def _stage1_body(F, BN, BE, fpA, fpB, W1cat, b1cat, esW2, esb2, evW2, evb2,
                 h_o, vx_o, vy_o, geo_o):
    fext = jnp.concatenate([fpA[...], fpB[...]], axis=0)
    M1 = BN + 1
    fi3 = jnp.broadcast_to(fext[:M1][:, None, :], (M1, K, F + 2))
    fj3 = jnp.concatenate(
        [fext[k + 1:k + 1 + M1][:, None, :] for k in range(K)], axis=1)
    fi = fi3.reshape(M1 * K, F + 2)
    fj = fj3.reshape(M1 * K, F + 2)
    pi = fi[:, F:]
    pj = fj[:, F:]
    rel = pj - pi
    sumsq = jnp.sum(rel * rel, axis=1, keepdims=True)
    inv = jax.lax.rsqrt(sumsq + 1e-12)
    length = jnp.sqrt(sumsq + 1e-12)
    dirv = rel * inv
    mp = 0.5 * (pi + pj)

    pd1 = mp[1:BE + 1] - mp[:BE]
    pd2 = mp[2:BE + 2] - mp[:BE]
    d1 = jnp.sqrt(jnp.sum(pd1 * pd1, axis=1, keepdims=True) + 1e-12)
    d2 = jnp.sqrt(jnp.sum(pd2 * pd2, axis=1, keepdims=True) + 1e-12)
    c1 = jnp.sum(dirv[1:BE + 1] * dirv[:BE], axis=1, keepdims=True)
    c2 = jnp.sum(dirv[2:BE + 2] * dirv[:BE], axis=1, keepdims=True)
    geo_o[...] = jnp.concatenate([d1, c1, d2, c2], axis=1)

    fiF = fi[:BE, :F]
    fjF = fj[:BE, :F]
    diff = jnp.abs(fiF - fjF)
    inputs = jnp.concatenate([fiF, fjF, diff, length[:BE]], axis=1)
    u = _silu(inputs @ W1cat[...] + b1cat[...])
    HID = esW2.shape[0]
    h = u[:, :HID] @ esW2[...] + esb2[...]
    amp = u[:, HID:] @ evW2[...] + evb2[...]
    h_o[...] = h
    vx_o[...] = amp * dirv[:BE, 0:1]
    vy_o[...] = amp * dirv[:BE, 1:2]


def _layer_body(BE, h_r, hn_r, vx_r, vxn_r, vy_r, vyn_r, geo_r,
                leW1, leb1, leW2, leb2, W1u, b1u, lnW2, lnb2, lvW2, lvb2,
                h_o, vx_o, vy_o):
    h = h_r[...]
    hn = hn_r[...]
    geo = geo_r[...]
    s1 = jnp.concatenate([h[1:], hn[:1]], axis=0)
    s2 = jnp.concatenate([h[2:], hn[:2]], axis=0)
    min1 = jnp.concatenate([s1, h, geo[:, 0:1], geo[:, 1:2]], axis=1)
    min2 = jnp.concatenate([s2, h, geo[:, 2:3], geo[:, 3:4]], axis=1)
    mi = jnp.concatenate([min1, min2], axis=0)
    m = _silu(mi @ leW1[...] + leb1[...]) @ leW2[...] + leb2[...]
    agg = m[:BE] + m[BE:]

    upd = jnp.concatenate([h, agg], axis=1)
    u = _silu(upd @ W1u[...] + b1u[...])
    HID = lnW2.shape[0]
    t = u[:, :HID] @ lnW2[...] + lnb2[...]
    coef = u[:, HID:] @ lvW2[...] + lvb2[...]
    h_o[...] = h + t

    vx = vx_r[...]
    vxn = vxn_r[...]
    vy = vy_r[...]
    vyn = vyn_r[...]
    vxagg = jnp.concatenate([vx[1:], vxn[:1]], axis=0) + \
        jnp.concatenate([vx[2:], vxn[:2]], axis=0)
    vyagg = jnp.concatenate([vy[1:], vyn[:1]], axis=0) + \
        jnp.concatenate([vy[2:], vyn[:2]], axis=0)
    vx_o[...] = vx + coef * vxagg
    vy_o[...] = vy + coef * vyagg


def _pick_bn(N):
    for bn in (200, 400, 80, 40, 16, 8):
        if N % bn == 0:
            return bn
    return N


def kernel(positions, features, es_W1, es_b1, es_W2, es_b2,
           ev_W1, ev_b1, ev_W2, ev_b2,
           le_W1, le_b1, le_W2, le_b2,
           ln_W1, ln_b1, ln_W2, ln_b2,
           lv_W1, lv_b1, lv_W2, lv_b2):
    N, F = features.shape
    E = N * K
    NL = le_W1.shape[0]
    HID = es_W1.shape[1]
    SD = es_W2.shape[1]
    VD = ev_W2.shape[1]
    BN = _pick_bn(N)
    nb = N // BN
    BE = BN * K

    fp = jnp.concatenate([features, positions], axis=1)
    W1cat = jnp.concatenate([es_W1, ev_W1], axis=1)
    b1cat = jnp.concatenate([es_b1, ev_b1])[None, :]

    full = lambda shape: pl.BlockSpec(shape, lambda i: tuple(0 for _ in shape))
    blk = lambda r, c: pl.BlockSpec((r, c), lambda i: (i, 0))
    nxt = lambda r, c: pl.BlockSpec((r, c), lambda i: ((i + 1) % nb, 0))

    h, vx, vy, geo = pl.pallas_call(
        functools.partial(_stage1_body, F, BN, BE),
        grid=(nb,),
        in_specs=[
            blk(BN, F + 2), nxt(BN, F + 2),
            full(W1cat.shape), full(b1cat.shape),
            full(es_W2.shape), full((1, SD)),
            full(ev_W2.shape), full((1, VD)),
        ],
        out_specs=[blk(BE, SD), blk(BE, VD), blk(BE, VD), blk(BE, 4)],
        out_shape=[
            jax.ShapeDtypeStruct((E, SD), jnp.float32),
            jax.ShapeDtypeStruct((E, VD), jnp.float32),
            jax.ShapeDtypeStruct((E, VD), jnp.float32),
            jax.ShapeDtypeStruct((E, 4), jnp.float32),
        ],
    )(fp, fp, W1cat, b1cat, es_W2, es_b2[None, :], ev_W2, ev_b2[None, :])

    layer_fn = pl.pallas_call(
        functools.partial(_layer_body, BE),
        grid=(nb,),
        in_specs=[
            blk(BE, SD), nxt(BE, SD),
            blk(BE, VD), nxt(BE, VD),
            blk(BE, VD), nxt(BE, VD),
            blk(BE, 4),
            full((2 * SD + 2, HID)), full((1, HID)),
            full((HID, SD)), full((1, SD)),
            full((2 * SD, 2 * HID)), full((1, 2 * HID)),
            full((HID, SD)), full((1, SD)),
            full((HID, VD)), full((1, VD)),
        ],
        out_specs=[blk(BE, SD), blk(BE, VD), blk(BE, VD)],
        out_shape=[
            jax.ShapeDtypeStruct((E, SD), jnp.float32),
            jax.ShapeDtypeStruct((E, VD), jnp.float32),
            jax.ShapeDtypeStruct((E, VD), jnp.float32),
        ],
    )

    for l in range(NL):
        W1u = jnp.concatenate([ln_W1[l], lv_W1[l]], axis=1)
        b1u = jnp.concatenate([ln_b1[l], lv_b1[l]])[None, :]
        h, vx, vy = layer_fn(
            h, h, vx, vx, vy, vy, geo,
            le_W1[l], le_b1[l][None, :], le_W2[l], le_b2[l][None, :],
            W1u, b1u, ln_W2[l], ln_b2[l][None, :], lv_W2[l], lv_b2[l][None, :])

    v = jnp.stack([vx, vy], axis=-1)
    return h, v


# speedup vs baseline: 94.0315x; 1.0580x over previous
import functools

import jax
import jax.numpy as jnp
from jax.experimental import pallas as pl
from jax.experimental.pallas import tpu as pltpu

K = 16
K2 = 2
BF = jnp.bfloat16
F32 = jnp.float32


def _dot(a, b):
    return jax.lax.dot_general(a, b, (((1,), (0,)), ((), ())),
                               preferred_element_type=F32)


def _bsilu(x):
    xb = x.astype(BF)
    return xb * jax.nn.sigmoid(xb)


def _stage1_body(F, BN, BE, SD, VD,
                 fA, fB, pA, pB, W1a, W1b, W1c, Wlen, b1, W2blk, b2,
                 h_o, vx_o, vy_o, geo_o, hh_o, vxh_o, vyh_o):
    fext = jnp.concatenate([fA[...], fB[...]], axis=0).astype(BF)
    pext = jnp.concatenate([pA[...], pB[...]], axis=0)
    M1 = BN + 1

    fi = jnp.broadcast_to(fext[:BN][:, None, :], (BN, K, F)).reshape(BE, F)
    fj = jnp.concatenate(
        [fext[k + 1:k + 1 + BN][:, None, :] for k in range(K)],
        axis=1).reshape(BE, F)
    diff = jnp.abs(fi - fj)

    piB = jnp.broadcast_to(pext[:M1][:, None, :],
                           (M1, K, 2 * VD)).reshape(M1 * K, 2 * VD)
    pjB = jnp.concatenate(
        [pext[k + 1:k + 1 + M1][:, None, :] for k in range(K)],
        axis=1).reshape(M1 * K, 2 * VD)
    relx = pjB[:, :VD] - piB[:, :VD]
    rely = pjB[:, VD:] - piB[:, VD:]
    ss = relx * relx + rely * rely
    inv = jax.lax.rsqrt(ss + 1e-12)
    len16 = ss * inv
    dirx = relx * inv
    diry = rely * inv
    mpB = 0.5 * (piB + pjB)

    pd1 = mpB[1:BE + 1] - mpB[:BE]
    pd2 = mpB[2:BE + 2] - mpB[:BE]
    d1 = jnp.sqrt(pd1[:, :VD] * pd1[:, :VD] + pd1[:, VD:] * pd1[:, VD:]
                  + 1e-12)
    d2 = jnp.sqrt(pd2[:, :VD] * pd2[:, :VD] + pd2[:, VD:] * pd2[:, VD:]
                  + 1e-12)
    c1 = dirx[1:BE + 1] * dirx[:BE] + diry[1:BE + 1] * diry[:BE]
    c2 = dirx[2:BE + 2] * dirx[:BE] + diry[2:BE + 2] * diry[:BE]
    geo_o[...] = jnp.concatenate(
        [d1[:, :1], c1[:, :1], d2[:, :1], c2[:, :1]], axis=1)

    pre = (_dot(fi, W1a[...]) + _dot(fj, W1b[...]) + _dot(diff, W1c[...])
           + _dot(len16[:BE].astype(BF), Wlen[...]) + b1[...])
    u = _bsilu(pre)
    r = _dot(u, W2blk[...]) + b2[...]
    h = r[:, :SD]
    amp = r[:, SD:]
    vx = amp * dirx[:BE]
    vy = amp * diry[:BE]
    h_o[...] = h
    vx_o[...] = vx
    vy_o[...] = vy
    hh_o[...] = h[:2][None]
    vxh_o[...] = vx[:2][None]
    vyh_o[...] = vy[:2][None]


def _layer_body(BE, SD, VD, HID,
                h_r, hh_r, vx_r, vxh_r, vy_r, vyh_r, geo_r,
                A, Bm, leb1, leW2, leb2x2, Wgeo, W1t, W1bot, b1u,
                lnW2, lnb2, lvW2, lvb2,
                h_o, vx_o, vy_o, hh_o, vxh_o, vyh_o):
    h = h_r[...]
    hb = h.astype(BF)
    hextb = jnp.concatenate([hb, hh_r[0].astype(BF)], axis=0)
    hA = _dot(hextb, A[...])
    s1 = hA[1:BE + 1]
    s2 = hA[2:BE + 2]
    hB = _dot(hb, Bm[...]) + leb1[...]
    g = _dot(geo_r[...].astype(BF), Wgeo[...])
    m1 = _bsilu(s1 + hB + g[:, :HID])
    m2 = _bsilu(s2 + hB + g[:, HID:])
    agg = _dot(m1, leW2[...]) + _dot(m2, leW2[...]) + leb2x2[...]

    u = _dot(hb, W1t[...]) + _dot(agg.astype(BF), W1bot[...]) + b1u[...]
    us = _bsilu(u)
    t = _dot(us[:, :HID], lnW2[...]) + lnb2[...]
    coef = _dot(us[:, HID:], lvW2[...]) + lvb2[...]
    hn = h + t
    h_o[...] = hn
    hh_o[...] = hn[:2][None]

    vx = vx_r[...]
    vy = vy_r[...]
    vxe = jnp.concatenate([vx, vxh_r[0]], axis=0)
    vye = jnp.concatenate([vy, vyh_r[0]], axis=0)
    vxn = vx + coef * (vxe[1:BE + 1] + vxe[2:BE + 2])
    vyn = vy + coef * (vye[1:BE + 1] + vye[2:BE + 2])
    vx_o[...] = vxn
    vy_o[...] = vyn
    vxh_o[...] = vxn[:2][None]
    vyh_o[...] = vyn[:2][None]


def _pick_bn(N):
    for bn in (400, 200, 80, 40, 16, 8):
        if N % bn == 0:
            return bn
    return N


def kernel(positions, features, es_W1, es_b1, es_W2, es_b2,
           ev_W1, ev_b1, ev_W2, ev_b2,
           le_W1, le_b1, le_W2, le_b2,
           ln_W1, ln_b1, ln_W2, ln_b2,
           lv_W1, lv_b1, lv_W2, lv_b2):
    N, F = features.shape
    E = N * K
    NL = le_W1.shape[0]
    HID = es_W1.shape[1]
    SD = es_W2.shape[1]
    VD = ev_W2.shape[1]
    BN = _pick_bn(N)
    nb = N // BN
    BE = BN * K

    posB = jnp.concatenate(
        [jnp.broadcast_to(positions[:, 0:1], (N, VD)),
         jnp.broadcast_to(positions[:, 1:2], (N, VD))], axis=1)

    W1es, W1ev = es_W1, ev_W1
    W1a = jnp.concatenate([W1es[:F], W1ev[:F]], axis=1).astype(BF)
    W1b = jnp.concatenate([W1es[F:2 * F], W1ev[F:2 * F]], axis=1).astype(BF)
    W1c = jnp.concatenate([W1es[2 * F:3 * F], W1ev[2 * F:3 * F]],
                          axis=1).astype(BF)
    wlen = jnp.concatenate([W1es[3 * F], W1ev[3 * F]])[None, :]
    Wlen = (jnp.ones((VD, 1), F32) / VD * wlen).astype(BF)
    b1cat = jnp.concatenate([es_b1, ev_b1])[None, :]
    W2blk = jnp.zeros((2 * HID, SD + VD), F32)
    W2blk = W2blk.at[:HID, :SD].set(es_W2).at[HID:, SD:].set(ev_W2).astype(BF)
    b2cat = jnp.concatenate([es_b2, ev_b2])[None, :]

    full = lambda shape: pl.BlockSpec(shape, lambda i: tuple(0 for _ in shape))
    blk = lambda r, c: pl.BlockSpec((r, c), lambda i: (i, 0))
    halo_in = lambda c: pl.BlockSpec((1, 2, c), lambda i: ((i + 1) % nb, 0, 0))
    halo_out = lambda c: pl.BlockSpec((1, 2, c), lambda i: (i, 0, 0))

    edge_out = lambda c: jax.ShapeDtypeStruct((E, c), F32)
    halo_shape = lambda c: jax.ShapeDtypeStruct((nb, 2, c), F32)

    h, vx, vy, geo, hh, vxh, vyh = pl.pallas_call(
        functools.partial(_stage1_body, F, BN, BE, SD, VD),
        grid=(nb,),
        in_specs=[
            blk(BN, F), pl.BlockSpec((BN, F), lambda i: ((i + 1) % nb, 0)),
            blk(BN, 2 * VD),
            pl.BlockSpec((BN, 2 * VD), lambda i: ((i + 1) % nb, 0)),
            full(W1a.shape), full(W1b.shape), full(W1c.shape),
            full(Wlen.shape), full(b1cat.shape),
            full(W2blk.shape), full(b2cat.shape),
        ],
        out_specs=[blk(BE, SD), blk(BE, VD), blk(BE, VD), blk(BE, 4),
                   halo_out(SD), halo_out(VD), halo_out(VD)],
        out_shape=[edge_out(SD), edge_out(VD), edge_out(VD), edge_out(4),
                   halo_shape(SD), halo_shape(VD), halo_shape(VD)],
    )(features, features, posB, posB, W1a, W1b, W1c, Wlen, b1cat, W2blk,
      b2cat)

    layer_fn = pl.pallas_call(
        functools.partial(_layer_body, BE, SD, VD, HID),
        grid=(nb,),
        in_specs=[
            blk(BE, SD), halo_in(SD),
            blk(BE, VD), halo_in(VD),
            blk(BE, VD), halo_in(VD),
            blk(BE, 4),
            full((SD, HID)), full((SD, HID)), full((1, HID)),
            full((HID, SD)), full((1, SD)), full((4, 2 * HID)),
            full((SD, 2 * HID)), full((SD, 2 * HID)), full((1, 2 * HID)),
            full((HID, SD)), full((1, SD)), full((HID, VD)), full((1, VD)),
        ],
        out_specs=[blk(BE, SD), blk(BE, VD), blk(BE, VD),
                   halo_out(SD), halo_out(VD), halo_out(VD)],
        out_shape=[edge_out(SD), edge_out(VD), edge_out(VD),
                   halo_shape(SD), halo_shape(VD), halo_shape(VD)],
    )

    for l in range(NL):
        A = le_W1[l][:SD].astype(BF)
        Bm = le_W1[l][SD:2 * SD].astype(BF)
        wd = le_W1[l][2 * SD]
        wc = le_W1[l][2 * SD + 1]
        Wgeo = jnp.zeros((4, 2 * HID), F32)
        Wgeo = (Wgeo.at[0, :HID].set(wd).at[1, :HID].set(wc)
                .at[2, HID:].set(wd).at[3, HID:].set(wc)).astype(BF)
        W1t = jnp.concatenate([ln_W1[l][:SD], lv_W1[l][:SD]],
                              axis=1).astype(BF)
        W1bot = jnp.concatenate([ln_W1[l][SD:], lv_W1[l][SD:]],
                                axis=1).astype(BF)
        b1u = jnp.concatenate([ln_b1[l], lv_b1[l]])[None, :]
        h, vx, vy, hh, vxh, vyh = layer_fn(
            h, hh, vx, vxh, vy, vyh, geo,
            A, Bm, le_b1[l][None, :], le_W2[l].astype(BF),
            (2.0 * le_b2[l])[None, :], Wgeo, W1t, W1bot, b1u,
            ln_W2[l].astype(BF), ln_b2[l][None, :],
            lv_W2[l].astype(BF), lv_b2[l][None, :])

    v = jnp.stack([vx, vy], axis=-1)
    return h, v


# speedup vs baseline: 111.4795x; 1.1856x over previous
import functools

import jax
import jax.numpy as jnp
from jax.experimental import pallas as pl
from jax.experimental.pallas import tpu as pltpu

K = 16
K2 = 2
BF = jnp.bfloat16
F32 = jnp.float32


def _dot(a, b, prec=F32):
    return jax.lax.dot_general(a, b, (((1,), (0,)), ((), ())),
                               preferred_element_type=prec)


def _silu(x):
    return x * (0.5 + 0.5 * jnp.tanh(0.5 * x))


def _geo_body(BNg, px_r, py_r, lenT_o, dirxT_o, diryT_o, geoP_o):
    px = jnp.concatenate([px_r[...], px_r[:, :K + 2]], axis=1)
    py = jnp.concatenate([py_r[...], py_r[:, :K + 2]], axis=1)
    M = BNg + 1
    pjx = jnp.concatenate([px[:, k + 1:k + 1 + M] for k in range(K)], axis=0)
    pjy = jnp.concatenate([py[:, k + 1:k + 1 + M] for k in range(K)], axis=0)
    pix = jnp.broadcast_to(px[:, :M], (K, M))
    piy = jnp.broadcast_to(py[:, :M], (K, M))
    relx = pjx - pix
    rely = pjy - piy
    ss = relx * relx + rely * rely
    inv = jax.lax.rsqrt(ss + 1e-12)
    lng = ss * inv
    dirx = relx * inv
    diry = rely * inv
    mpx = 0.5 * (pix + pjx)
    mpy = 0.5 * (piy + pjy)

    def s1(X):
        return jnp.concatenate([X[1:, :BNg], X[0:1, 1:BNg + 1]], axis=0)

    def s2(X):
        return jnp.concatenate([X[2:, :BNg], X[0:2, 1:BNg + 1]], axis=0)

    dx1 = s1(mpx) - mpx[:, :BNg]
    dy1 = s1(mpy) - mpy[:, :BNg]
    dx2 = s2(mpx) - mpx[:, :BNg]
    dy2 = s2(mpy) - mpy[:, :BNg]
    d1 = jnp.sqrt(dx1 * dx1 + dy1 * dy1 + 1e-12)
    d2 = jnp.sqrt(dx2 * dx2 + dy2 * dy2 + 1e-12)
    c1 = s1(dirx) * dirx[:, :BNg] + s1(diry) * diry[:, :BNg]
    c2 = s2(dirx) * dirx[:, :BNg] + s2(diry) * diry[:, :BNg]

    tr = lambda x: jnp.transpose(x, (1, 0)).astype(BF)
    lenT_o[...] = tr(lng[:, :BNg])
    dirxT_o[...] = tr(dirx[:, :BNg])
    diryT_o[...] = tr(diry[:, :BNg])
    geoP_o[...] = jnp.concatenate([tr(d1), tr(c1), tr(d2), tr(c2)], axis=1)


def _stage1_body(F, BN, BE, SD, VD,
                 fA, fB, lenT, dxyT, W1a, W1b, W1c, WlenB, b1, W2blk, b2,
                 DB2, h_o, vx_o, vy_o, hh_o, vxh_o, vyh_o):
    fext = jnp.concatenate([fA[...], fB[...]], axis=0).astype(BF)
    fi = jnp.broadcast_to(fext[:BN][:, None, :], (BN, K, F)).reshape(BE, F)
    fj = jnp.concatenate(
        [fext[k + 1:k + 1 + BN][:, None, :] for k in range(K)],
        axis=1).reshape(BE, F)
    diff = jnp.abs(fi - fj)

    lenadd = _dot(lenT[...], WlenB[...]).astype(BF).reshape(BE, 128)
    pre = (_dot(fi, W1a[...]) + _dot(fj, W1b[...]) + _dot(diff, W1c[...])
           + lenadd + b1[...])
    u = _silu(pre).astype(BF)
    r = _dot(u, W2blk[...]) + b2[...]
    h = r[:, :SD]
    amp = r[:, SD:]

    dxy = _dot(dxyT[...], DB2[...]).astype(BF).reshape(BE, 128)
    vx = amp * dxy[:, :VD]
    vy = amp * dxy[:, VD:2 * VD]
    h_o[...] = h
    vx_o[...] = vx
    vy_o[...] = vy
    hh_o[...] = h[:2][None]
    vxh_o[...] = vx[:2][None]
    vyh_o[...] = vy[:2][None]


def _layer_body(BE, SD, VD, HID,
                h_r, hh_r, vx_r, vxh_r, vy_r, vyh_r, geoP_r,
                A, Bm, leb1, leW2, leb2x2, WgeoB, W1t, W1bot, b1u,
                lnW2, lnb2, lvW2, lvb2,
                h_o, vx_o, vy_o, hh_o, vxh_o, vyh_o):
    h = h_r[...]
    hb = h.astype(BF)
    hextb = jnp.concatenate([hb, hh_r[0].astype(BF)], axis=0)
    hA = _dot(hextb, A[...])
    s1 = hA[1:BE + 1]
    s2 = hA[2:BE + 2]
    hB = _dot(hb, Bm[...]) + leb1[...]
    g = _dot(geoP_r[...], WgeoB[...]).astype(BF).reshape(BE, 128)
    m1 = _silu(s1 + hB + g[:, :HID]).astype(BF)
    m2 = _silu(s2 + hB + g[:, HID:]).astype(BF)
    agg = _dot(m1, leW2[...]) + _dot(m2, leW2[...]) + leb2x2[...]

    u = _dot(hb, W1t[...]) + _dot(agg.astype(BF), W1bot[...]) + b1u[...]
    us = _silu(u).astype(BF)
    t = _dot(us[:, :HID], lnW2[...]) + lnb2[...]
    coef = _dot(us[:, HID:], lvW2[...]) + lvb2[...]
    hn = h + t
    h_o[...] = hn
    hh_o[...] = hn[:2][None]

    vx = vx_r[...]
    vy = vy_r[...]
    vxe = jnp.concatenate([vx, vxh_r[0]], axis=0)
    vye = jnp.concatenate([vy, vyh_r[0]], axis=0)
    vxn = vx + coef * (vxe[1:BE + 1] + vxe[2:BE + 2])
    vyn = vy + coef * (vye[1:BE + 1] + vye[2:BE + 2])
    vx_o[...] = vxn
    vy_o[...] = vyn
    vxh_o[...] = vxn[:2][None]
    vyh_o[...] = vyn[:2][None]


def _pick(N, cands):
    for c in cands:
        if N % c == 0 and c <= N:
            return c
    return N


def kernel(positions, features, es_W1, es_b1, es_W2, es_b2,
           ev_W1, ev_b1, ev_W2, ev_b2,
           le_W1, le_b1, le_W2, le_b2,
           ln_W1, ln_b1, ln_W2, ln_b2,
           lv_W1, lv_b1, lv_W2, lv_b2):
    N, F = features.shape
    E = N * K
    NL = le_W1.shape[0]
    HID = es_W1.shape[1]
    SD = es_W2.shape[1]
    VD = ev_W2.shape[1]
    BN = _pick(N, (400, 200, 80, 40, 16, 8))
    nb = N // BN
    BE = BN * K
    BNg = N

    eye = jnp.eye(K, dtype=F32)
    z64 = jnp.zeros((HID,), F32)
    onesV = jnp.ones((VD,), F32)
    zpad = jnp.zeros((128 - 2 * VD,), F32)

    wlen_cat = jnp.concatenate([es_W1[3 * F], ev_W1[3 * F]])
    WlenB = jnp.kron(eye, wlen_cat[None, :]).astype(BF)
    DB2 = jnp.concatenate([
        jnp.kron(eye, jnp.concatenate([onesV, jnp.zeros((VD,), F32),
                                       zpad])[None, :]),
        jnp.kron(eye, jnp.concatenate([jnp.zeros((VD,), F32), onesV,
                                       zpad])[None, :]),
    ], axis=0).astype(BF)

    W1a = jnp.concatenate([es_W1[:F], ev_W1[:F]], axis=1).astype(BF)
    W1b = jnp.concatenate([es_W1[F:2 * F], ev_W1[F:2 * F]], axis=1).astype(BF)
    W1c = jnp.concatenate([es_W1[2 * F:3 * F], ev_W1[2 * F:3 * F]],
                          axis=1).astype(BF)
    b1cat = jnp.concatenate([es_b1, ev_b1])[None, :]
    W2blk = jnp.zeros((2 * HID, SD + VD), F32)
    W2blk = W2blk.at[:HID, :SD].set(es_W2).at[HID:, SD:].set(ev_W2).astype(BF)
    b2cat = jnp.concatenate([es_b2, ev_b2])[None, :]

    posT = positions.T
    posx = posT[0:1]
    posy = posT[1:2]

    full = lambda shape: pl.BlockSpec(shape, lambda i: tuple(0 for _ in shape))
    blk = lambda r, c: pl.BlockSpec((r, c), lambda i: (i, 0))
    halo_in = lambda c: pl.BlockSpec((1, 2, c), lambda i: ((i + 1) % nb, 0, 0))
    halo_out = lambda c: pl.BlockSpec((1, 2, c), lambda i: (i, 0, 0))
    edge_out = lambda c: jax.ShapeDtypeStruct((E, c), F32)
    halo_shape = lambda c: jax.ShapeDtypeStruct((nb, 2, c), F32)

    lenT, dirxT, diryT, geoP = pl.pallas_call(
        functools.partial(_geo_body, BNg),
        grid=(1,),
        in_specs=[full((1, N)), full((1, N))],
        out_specs=[full((N, K)), full((N, K)), full((N, K)),
                   full((N, 4 * K))],
        out_shape=[jax.ShapeDtypeStruct((N, K), BF),
                   jax.ShapeDtypeStruct((N, K), BF),
                   jax.ShapeDtypeStruct((N, K), BF),
                   jax.ShapeDtypeStruct((N, 4 * K), BF)],
    )(posx, posy)

    dxyT = jnp.concatenate([dirxT, diryT], axis=1)

    h, vx, vy, hh, vxh, vyh = pl.pallas_call(
        functools.partial(_stage1_body, F, BN, BE, SD, VD),
        grid=(nb,),
        in_specs=[
            blk(BN, F), pl.BlockSpec((BN, F), lambda i: ((i + 1) % nb, 0)),
            blk(BN, K), blk(BN, 2 * K),
            full(W1a.shape), full(W1b.shape), full(W1c.shape),
            full(WlenB.shape), full(b1cat.shape),
            full(W2blk.shape), full(b2cat.shape), full(DB2.shape),
        ],
        out_specs=[blk(BE, SD), blk(BE, VD), blk(BE, VD),
                   halo_out(SD), halo_out(VD), halo_out(VD)],
        out_shape=[edge_out(SD), edge_out(VD), edge_out(VD),
                   halo_shape(SD), halo_shape(VD), halo_shape(VD)],
    )(features, features, lenT, dxyT, W1a, W1b, W1c, WlenB, b1cat, W2blk,
      b2cat, DB2)

    layer_fn = pl.pallas_call(
        functools.partial(_layer_body, BE, SD, VD, HID),
        grid=(nb,),
        in_specs=[
            blk(BE, SD), halo_in(SD),
            blk(BE, VD), halo_in(VD),
            blk(BE, VD), halo_in(VD),
            blk(BN, 4 * K),
            full((SD, HID)), full((SD, HID)), full((1, HID)),
            full((HID, SD)), full((1, SD)), full((4 * K, 2 * HID * K)),
            full((SD, 2 * HID)), full((SD, 2 * HID)), full((1, 2 * HID)),
            full((HID, SD)), full((1, SD)), full((HID, VD)), full((1, VD)),
        ],
        out_specs=[blk(BE, SD), blk(BE, VD), blk(BE, VD),
                   halo_out(SD), halo_out(VD), halo_out(VD)],
        out_shape=[edge_out(SD), edge_out(VD), edge_out(VD),
                   halo_shape(SD), halo_shape(VD), halo_shape(VD)],
    )

    zH = jnp.zeros((HID,), F32)
    for l in range(NL):
        A = le_W1[l][:SD].astype(BF)
        Bm = le_W1[l][SD:2 * SD].astype(BF)
        wd = le_W1[l][2 * SD]
        wc = le_W1[l][2 * SD + 1]
        WgeoB = jnp.concatenate([
            jnp.kron(eye, jnp.concatenate([wd, zH])[None, :]),
            jnp.kron(eye, jnp.concatenate([wc, zH])[None, :]),
            jnp.kron(eye, jnp.concatenate([zH, wd])[None, :]),
            jnp.kron(eye, jnp.concatenate([zH, wc])[None, :]),
        ], axis=0).astype(BF)
        W1t = jnp.concatenate([ln_W1[l][:SD], lv_W1[l][:SD]],
                              axis=1).astype(BF)
        W1bot = jnp.concatenate([ln_W1[l][SD:], lv_W1[l][SD:]],
                                axis=1).astype(BF)
        b1u = jnp.concatenate([ln_b1[l], lv_b1[l]])[None, :]
        h, vx, vy, hh, vxh, vyh = layer_fn(
            h, hh, vx, vxh, vy, vyh, geoP,
            A, Bm, le_b1[l][None, :], le_W2[l].astype(BF),
            (2.0 * le_b2[l])[None, :], WgeoB, W1t, W1bot, b1u,
            ln_W2[l].astype(BF), ln_b2[l][None, :],
            lv_W2[l].astype(BF), lv_b2[l][None, :])

    v = jnp.stack([vx, vy], axis=-1)
    return h, v
